# baseline (speedup 1.0000x reference)
import math

import jax
import jax.numpy as jnp
from jax.experimental import pallas as pl
from jax.experimental.pallas import tpu as pltpu

_TARGET_STEPS = (7, 15, 23, 31)

_TWO_OVER_PI = 0.63661975
_PIO2_H = 1.5705566
_PIO2_M = 0.00023968617


def _sincos(x):
    ki = jnp.round(x * _TWO_OVER_PI).astype(jnp.int32)
    kf = ki.astype(jnp.float32)
    r = (x - kf * _PIO2_H) - kf * _PIO2_M
    z = r * r
    sp = ((-1.9515295891e-4 * z + 8.3321608736e-3) * z - 1.6666654611e-1)
    s = r + r * z * sp
    cp = ((2.443315711809948e-5 * z - 1.388731625493765e-3) * z
          + 4.166664568298827e-2)
    c = 1.0 - 0.5 * z + z * z * cp
    swap = (ki & 1) == 1
    s_sel = jnp.where(swap, c, s)
    c_sel = jnp.where(swap, s, c)
    s_signbit = jax.lax.shift_left(ki & 2, 30)
    c_signbit = jax.lax.shift_left((ki + 1) & 2, 30)
    bc = jax.lax.bitcast_convert_type
    sinx = bc(bc(s_sel, jnp.int32) ^ s_signbit, jnp.float32)
    cosx = bc(bc(c_sel, jnp.int32) ^ c_signbit, jnp.float32)
    return sinx, cosx
_POS_W, _ORI_W, _POSE_W = 1.0, 0.5, 1.0
_LIMIT_W, _REST_W = 10.0, 0.1
_BASE_POS_W, _BASE_ORI_W = 5.0, 5.0


def _rotmat(sx, sy, sz, sw):
    xx, yy, zz = sx * sx, sy * sy, sz * sz
    xy, xz, yz = sx * sy, sx * sz, sy * sz
    wx, wy, wz = sw * sx, sw * sy, sw * sz
    return ((1.0 - 2.0 * (yy + zz), 2.0 * (xy - wz), 2.0 * (xz + wy)),
            (2.0 * (xy + wz), 1.0 - 2.0 * (xx + zz), 2.0 * (yz - wx)),
            (2.0 * (xz - wy), 2.0 * (yz + wx), 1.0 - 2.0 * (xx + yy)))


def _se3_log(tx, ty, tz, qx, qy, qz, qw):
    neg = qw < 0.0
    qx = jnp.where(neg, -qx, qx)
    qy = jnp.where(neg, -qy, qy)
    qz = jnp.where(neg, -qz, qz)
    qw = jnp.where(neg, -qw, qw)
    nv2 = qx * qx + qy * qy + qz * qz
    nv = jnp.sqrt(jnp.maximum(nv2, 1e-14))
    angle = 2.0 * jnp.arctan2(nv, qw)
    small_q = nv < 1e-6
    w_safe = jnp.where(jnp.abs(qw) > 1e-8, qw, 1.0)
    scale = jnp.where(small_q, 2.0 / w_safe, angle / nv)
    px, py, pz = qx * scale, qy * scale, qz * scale
    th2 = px * px + py * py + pz * pz
    th = jnp.sqrt(jnp.maximum(th2, 1e-14))
    small_t = th < 1e-4
    a = jnp.where(small_t,
                  1.0 / 12.0 + th2 / 720.0,
                  (1.0 - 0.5 * th * qw / nv) / th2)
    ptx = py * tz - pz * ty
    pty = pz * tx - px * tz
    ptz = px * ty - py * tx
    p2x = py * ptz - pz * pty
    p2y = pz * ptx - px * ptz
    p2z = px * pty - py * ptx
    rx = tx - 0.5 * ptx + a * p2x
    ry = ty - 0.5 * pty + a * p2y
    rz = tz - 0.5 * ptz + a * p2z
    return rx, ry, rz, px, py, pz


def _pose_residual(tx, ty, tz, qx, qy, qz, qw, st, sq):
    iqx, iqy, iqz, iqw = -sq[0], -sq[1], -sq[2], sq[3]
    R = _rotmat(iqx, iqy, iqz, iqw)
    dx, dy, dz = tx - st[0], ty - st[1], tz - st[2]
    ex = R[0][0] * dx + R[0][1] * dy + R[0][2] * dz
    ey = R[1][0] * dx + R[1][1] * dy + R[1][2] * dz
    ez = R[2][0] * dx + R[2][1] * dy + R[2][2] * dz
    ew = iqw * qw - iqx * qx - iqy * qy - iqz * qz
    evx = iqw * qx + qw * iqx + (iqy * qz - iqz * qy)
    evy = iqw * qy + qw * iqy + (iqz * qx - iqx * qz)
    evz = iqw * qz + qw * iqz + (iqx * qy - iqy * qx)
    return _se3_log(ex, ey, ez, evx, evy, evz, ew)


def _natural_store(comps, out_ref):
    ncols = out_ref.shape[1]
    npad = (ncols + 7) // 8 * 8
    ngrp = npad // 8
    bbr = comps[0].shape[0]
    zero = jnp.zeros_like(comps[0])
    comps = comps + [zero] * (npad - ncols)
    iota = jax.lax.broadcasted_iota(jnp.int32, (8, 128), 0)
    for tr in range(bbr // 8):
        cols = []
        for g in range(ngrp):
            x = [comps[8 * g + i][8 * tr:8 * tr + 8, :] for i in range(8)]
            for k in (1, 2, 4):
                xn = []
                for i in range(8):
                    sh = k if (i & k) == 0 else 8 - k
                    rolled = pltpu.roll(x[i ^ k], sh, 0)
                    xn.append(jnp.where((iota & k) == (i & k), x[i], rolled))
                x = xn
            cols.append(x)
        for i in range(8):
            s_blk = jnp.concatenate([cols[g][i] for g in range(ngrp)], axis=0)
            s_t = jnp.swapaxes(s_blk, 0, 1)
            r = 8 * tr + i
            out_ref[r * 128:(r + 1) * 128, :] = s_t[:, :ncols]


def _ik_kernel(cfg_ref, base_ref, tgt_ref, dflt_ref, rest_ref, off_ref,
               ax_ref, lo_ref, up_ref, out_ref):
    nj = cfg_ref.shape[0]
    out_cols = [None] * out_ref.shape[1]

    btx, bty, btz = base_ref[0], base_ref[1], base_ref[2]
    bqx, bqy, bqz, bqw = base_ref[3], base_ref[4], base_ref[5], base_ref[6]

    d_t = (dflt_ref[0], dflt_ref[1], dflt_ref[2])
    d_q = (dflt_ref[3], dflt_ref[4], dflt_ref[5], dflt_ref[6])
    brx, bry, brz, bpx, bpy, bpz = _pose_residual(
        btx, bty, btz, bqx, bqy, bqz, bqw, d_t, d_q)
    out_cols[88] = brx * _BASE_POS_W
    out_cols[89] = bry * _BASE_POS_W
    out_cols[90] = brz * _BASE_POS_W
    out_cols[91] = bpx * _BASE_ORI_W
    out_cols[92] = bpy * _BASE_ORI_W
    out_cols[93] = bpz * _BASE_ORI_W

    for j in range(nj):
        cj = cfg_ref[j]
        lo, up, rs = lo_ref[j], up_ref[j], rest_ref[j]
        out_cols[24 + j] = (cj - jnp.minimum(jnp.maximum(cj, lo), up)) * _LIMIT_W
        out_cols[56 + j] = (cj - rs) * _REST_W

    tx, ty, tz = btx, bty, btz
    qx, qy, qz, qw = bqx, bqy, bqz, bqw
    tgt_idx = 0
    for j in range(nj):
        otx, oty, otz = off_ref[j, 0], off_ref[j, 1], off_ref[j, 2]
        oqx, oqy, oqz, oqw = off_ref[j, 3], off_ref[j, 4], off_ref[j, 5], off_ref[j, 6]
        ux = qy * otz - qz * oty + qw * otx
        uy = qz * otx - qx * otz + qw * oty
        uz = qx * oty - qy * otx + qw * otz
        tx = tx + otx + 2.0 * (qy * uz - qz * uy)
        ty = ty + oty + 2.0 * (qz * ux - qx * uz)
        tz = tz + otz + 2.0 * (qx * uy - qy * ux)
        axx, axy, axz = ax_ref[j, 0], ax_ref[j, 1], ax_ref[j, 2]
        p_x = oqw * axx + (oqy * axz - oqz * axy)
        p_y = oqw * axy + (oqz * axx - oqx * axz)
        p_z = oqw * axz + (oqx * axy - oqy * axx)
        dd = oqx * axx + oqy * axy + oqz * axz
        half = 0.5 * cfg_ref[j]
        s, c = _sincos(half)
        mx = s * p_x + c * oqx
        my = s * p_y + c * oqy
        mz = s * p_z + c * oqz
        mw = c * oqw - s * dd
        nw = qw * mw - qx * mx - qy * my - qz * mz
        nx = qw * mx + mw * qx + (qy * mz - qz * my)
        ny = qw * my + mw * qy + (qz * mx - qx * mz)
        nz = qw * mz + mw * qz + (qx * my - qy * mx)
        qx, qy, qz, qw = nx, ny, nz, nw

        if j in _TARGET_STEPS:
            k = tgt_idx
            tgt_idx += 1
            s_t = (tgt_ref[k, 0], tgt_ref[k, 1], tgt_ref[k, 2])
            s_q = (tgt_ref[k, 3], tgt_ref[k, 4], tgt_ref[k, 5], tgt_ref[k, 6])
            rx, ry, rz, ppx, ppy, ppz = _pose_residual(
                tx, ty, tz, qx, qy, qz, qw, s_t, s_q)
            pw = _POS_W * _POSE_W
            ow = _ORI_W * _POSE_W
            out_cols[6 * k + 0] = rx * pw
            out_cols[6 * k + 1] = ry * pw
            out_cols[6 * k + 2] = rz * pw
            out_cols[6 * k + 3] = ppx * ow
            out_cols[6 * k + 4] = ppy * ow
            out_cols[6 * k + 5] = ppz * ow

    _natural_store(out_cols, out_ref)


def _run_half(cfg, base, target_poses, default_base, rest_cfg,
              joint_offsets, joint_axes, joint_lower, joint_upper):
    b, nj = cfg.shape
    nt = target_poses.shape[0]
    ncols = 6 * nt + 2 * nj + 6
    ncols_pad = (ncols + 7) // 8 * 8
    r = b // 128
    grid = math.gcd(r, 16)
    bbr = r // grid

    cfg_t = cfg.T.reshape(nj, r, 128)
    base_t = base.T.reshape(7, r, 128)

    smem = pl.BlockSpec(memory_space=pltpu.SMEM)
    out = pl.pallas_call(
        _ik_kernel,
        out_shape=jax.ShapeDtypeStruct((b, ncols), jnp.float32),
        grid=(grid,),
        in_specs=[
            pl.BlockSpec((nj, bbr, 128), lambda i: (0, i, 0)),
            pl.BlockSpec((7, bbr, 128), lambda i: (0, i, 0)),
            smem,
            smem,
            smem,
            smem,
            smem,
            smem,
            smem,
        ],
        out_specs=pl.BlockSpec((bbr * 128, ncols), lambda i: (i, 0)),
        compiler_params=pltpu.CompilerParams(
            dimension_semantics=("parallel",),
        ),
        name="floating_base_ik",
    )(cfg_t, base_t, target_poses, default_base, rest_cfg,
      joint_offsets, joint_axes, joint_lower, joint_upper)

    return out


def kernel(cfg, base, target_poses, default_base, rest_cfg,
           joint_offsets, joint_axes, joint_lower, joint_upper):
    return _run_half(cfg, base, target_poses, default_base, rest_cfg,
                     joint_offsets, joint_axes, joint_lower, joint_upper)


# speedup vs baseline: 1.1740x; 1.1740x over previous
import math

import jax
import jax.numpy as jnp
from jax.experimental import pallas as pl
from jax.experimental.pallas import tpu as pltpu

_TARGET_STEPS = (7, 15, 23, 31)

_TWO_OVER_PI = 0.63661975
_PIO2_H = 1.5705566
_PIO2_M = 0.00023968617


def _sincos(x):
    ki = jnp.round(x * _TWO_OVER_PI).astype(jnp.int32)
    kf = ki.astype(jnp.float32)
    r = (x - kf * _PIO2_H) - kf * _PIO2_M
    z = r * r
    sp = ((-1.9515295891e-4 * z + 8.3321608736e-3) * z - 1.6666654611e-1)
    s = r + r * z * sp
    cp = ((2.443315711809948e-5 * z - 1.388731625493765e-3) * z
          + 4.166664568298827e-2)
    c = 1.0 - 0.5 * z + z * z * cp
    swap = (ki & 1) == 1
    s_sel = jnp.where(swap, c, s)
    c_sel = jnp.where(swap, s, c)
    s_signbit = jax.lax.shift_left(ki & 2, 30)
    c_signbit = jax.lax.shift_left((ki + 1) & 2, 30)
    bc = jax.lax.bitcast_convert_type
    sinx = bc(bc(s_sel, jnp.int32) ^ s_signbit, jnp.float32)
    cosx = bc(bc(c_sel, jnp.int32) ^ c_signbit, jnp.float32)
    return sinx, cosx
_POS_W, _ORI_W, _POSE_W = 1.0, 0.5, 1.0
_LIMIT_W, _REST_W = 10.0, 0.1
_BASE_POS_W, _BASE_ORI_W = 5.0, 5.0


def _rotmat(sx, sy, sz, sw):
    xx, yy, zz = sx * sx, sy * sy, sz * sz
    xy, xz, yz = sx * sy, sx * sz, sy * sz
    wx, wy, wz = sw * sx, sw * sy, sw * sz
    return ((1.0 - 2.0 * (yy + zz), 2.0 * (xy - wz), 2.0 * (xz + wy)),
            (2.0 * (xy + wz), 1.0 - 2.0 * (xx + zz), 2.0 * (yz - wx)),
            (2.0 * (xz - wy), 2.0 * (yz + wx), 1.0 - 2.0 * (xx + yy)))


def _se3_log(tx, ty, tz, qx, qy, qz, qw):
    neg = qw < 0.0
    qx = jnp.where(neg, -qx, qx)
    qy = jnp.where(neg, -qy, qy)
    qz = jnp.where(neg, -qz, qz)
    qw = jnp.where(neg, -qw, qw)
    nv2 = qx * qx + qy * qy + qz * qz
    nv = jnp.sqrt(jnp.maximum(nv2, 1e-14))
    angle = 2.0 * jnp.arctan2(nv, qw)
    small_q = nv < 1e-6
    w_safe = jnp.where(jnp.abs(qw) > 1e-8, qw, 1.0)
    scale = jnp.where(small_q, 2.0 / w_safe, angle / nv)
    px, py, pz = qx * scale, qy * scale, qz * scale
    th2 = px * px + py * py + pz * pz
    th = jnp.sqrt(jnp.maximum(th2, 1e-14))
    small_t = th < 1e-4
    a = jnp.where(small_t,
                  1.0 / 12.0 + th2 / 720.0,
                  (1.0 - 0.5 * th * qw / nv) / th2)
    ptx = py * tz - pz * ty
    pty = pz * tx - px * tz
    ptz = px * ty - py * tx
    p2x = py * ptz - pz * pty
    p2y = pz * ptx - px * ptz
    p2z = px * pty - py * ptx
    rx = tx - 0.5 * ptx + a * p2x
    ry = ty - 0.5 * pty + a * p2y
    rz = tz - 0.5 * ptz + a * p2z
    return rx, ry, rz, px, py, pz


def _pose_residual(tx, ty, tz, qx, qy, qz, qw, st, sq):
    iqx, iqy, iqz, iqw = -sq[0], -sq[1], -sq[2], sq[3]
    R = _rotmat(iqx, iqy, iqz, iqw)
    dx, dy, dz = tx - st[0], ty - st[1], tz - st[2]
    ex = R[0][0] * dx + R[0][1] * dy + R[0][2] * dz
    ey = R[1][0] * dx + R[1][1] * dy + R[1][2] * dz
    ez = R[2][0] * dx + R[2][1] * dy + R[2][2] * dz
    ew = iqw * qw - iqx * qx - iqy * qy - iqz * qz
    evx = iqw * qx + qw * iqx + (iqy * qz - iqz * qy)
    evy = iqw * qy + qw * iqy + (iqz * qx - iqx * qz)
    evz = iqw * qz + qw * iqz + (iqx * qy - iqy * qx)
    return _se3_log(ex, ey, ez, evx, evy, evz, ew)


def _ik_kernel(cfg_ref, base_ref, tgt_ref, dflt_ref, rest_ref, off_ref,
               ax_ref, lo_ref, up_ref, out_ref):
    nj = cfg_ref.shape[0]

    btx, bty, btz = base_ref[0], base_ref[1], base_ref[2]
    bqx, bqy, bqz, bqw = base_ref[3], base_ref[4], base_ref[5], base_ref[6]

    d_t = (dflt_ref[0], dflt_ref[1], dflt_ref[2])
    d_q = (dflt_ref[3], dflt_ref[4], dflt_ref[5], dflt_ref[6])
    brx, bry, brz, bpx, bpy, bpz = _pose_residual(
        btx, bty, btz, bqx, bqy, bqz, bqw, d_t, d_q)
    out_ref[88] = brx * _BASE_POS_W
    out_ref[89] = bry * _BASE_POS_W
    out_ref[90] = brz * _BASE_POS_W
    out_ref[91] = bpx * _BASE_ORI_W
    out_ref[92] = bpy * _BASE_ORI_W
    out_ref[93] = bpz * _BASE_ORI_W

    for j in range(nj):
        cj = cfg_ref[j]
        lo, up, rs = lo_ref[j], up_ref[j], rest_ref[j]
        out_ref[24 + j] = (cj - jnp.minimum(jnp.maximum(cj, lo), up)) * _LIMIT_W
        out_ref[56 + j] = (cj - rs) * _REST_W

    tx, ty, tz = btx, bty, btz
    qx, qy, qz, qw = bqx, bqy, bqz, bqw
    tgt_idx = 0
    for j in range(nj):
        otx, oty, otz = off_ref[j, 0], off_ref[j, 1], off_ref[j, 2]
        oqx, oqy, oqz, oqw = off_ref[j, 3], off_ref[j, 4], off_ref[j, 5], off_ref[j, 6]
        ux = qy * otz - qz * oty + qw * otx
        uy = qz * otx - qx * otz + qw * oty
        uz = qx * oty - qy * otx + qw * otz
        tx = tx + otx + 2.0 * (qy * uz - qz * uy)
        ty = ty + oty + 2.0 * (qz * ux - qx * uz)
        tz = tz + otz + 2.0 * (qx * uy - qy * ux)
        axx, axy, axz = ax_ref[j, 0], ax_ref[j, 1], ax_ref[j, 2]
        p_x = oqw * axx + (oqy * axz - oqz * axy)
        p_y = oqw * axy + (oqz * axx - oqx * axz)
        p_z = oqw * axz + (oqx * axy - oqy * axx)
        dd = oqx * axx + oqy * axy + oqz * axz
        half = 0.5 * cfg_ref[j]
        s, c = _sincos(half)
        mx = s * p_x + c * oqx
        my = s * p_y + c * oqy
        mz = s * p_z + c * oqz
        mw = c * oqw - s * dd
        nw = qw * mw - qx * mx - qy * my - qz * mz
        nx = qw * mx + mw * qx + (qy * mz - qz * my)
        ny = qw * my + mw * qy + (qz * mx - qx * mz)
        nz = qw * mz + mw * qz + (qx * my - qy * mx)
        qx, qy, qz, qw = nx, ny, nz, nw

        if j in _TARGET_STEPS:
            k = tgt_idx
            tgt_idx += 1
            s_t = (tgt_ref[k, 0], tgt_ref[k, 1], tgt_ref[k, 2])
            s_q = (tgt_ref[k, 3], tgt_ref[k, 4], tgt_ref[k, 5], tgt_ref[k, 6])
            rx, ry, rz, ppx, ppy, ppz = _pose_residual(
                tx, ty, tz, qx, qy, qz, qw, s_t, s_q)
            pw = _POS_W * _POSE_W
            ow = _ORI_W * _POSE_W
            out_ref[6 * k + 0] = rx * pw
            out_ref[6 * k + 1] = ry * pw
            out_ref[6 * k + 2] = rz * pw
            out_ref[6 * k + 3] = ppx * ow
            out_ref[6 * k + 4] = ppy * ow
            out_ref[6 * k + 5] = ppz * ow


def _run_half(cfg, base, target_poses, default_base, rest_cfg,
              joint_offsets, joint_axes, joint_lower, joint_upper):
    b, nj = cfg.shape
    nt = target_poses.shape[0]
    ncols = 6 * nt + 2 * nj + 6
    ncols_pad = (ncols + 7) // 8 * 8
    r = b // 128
    grid = math.gcd(r, 16)
    bbr = r // grid

    cfg_t = cfg.T.reshape(nj, r, 128)
    base_t = base.T.reshape(7, r, 128)

    smem = pl.BlockSpec(memory_space=pltpu.SMEM)
    out = pl.pallas_call(
        _ik_kernel,
        out_shape=jax.ShapeDtypeStruct((ncols_pad, r, 128), jnp.float32),
        grid=(grid,),
        in_specs=[
            pl.BlockSpec((nj, bbr, 128), lambda i: (0, i, 0)),
            pl.BlockSpec((7, bbr, 128), lambda i: (0, i, 0)),
            smem,
            smem,
            smem,
            smem,
            smem,
            smem,
            smem,
        ],
        out_specs=pl.BlockSpec((ncols_pad, bbr, 128), lambda i: (0, i, 0)),
        compiler_params=pltpu.CompilerParams(
            dimension_semantics=("parallel",),
        ),
        name="floating_base_ik",
    )(cfg_t, base_t, target_poses, default_base, rest_cfg,
      joint_offsets, joint_axes, joint_lower, joint_upper)

    return out.reshape(ncols_pad, b).T[:, :ncols]


def kernel(cfg, base, target_poses, default_base, rest_cfg,
           joint_offsets, joint_axes, joint_lower, joint_upper):
    return _run_half(cfg, base, target_poses, default_base, rest_cfg,
                     joint_offsets, joint_axes, joint_lower, joint_upper)
